# baseline (speedup 1.0000x reference)
import jax
import jax.numpy as jnp
from jax.experimental import pallas as pl
from jax.experimental.pallas import tpu as pltpu

_B, _C, _H, _W = 16, 256, 32, 32


_KREP = 2


def _body(row_ref, col_ref, out_ref, scratch, sems):
    col_b = jnp.broadcast_to(col_ref[...][None, :, :], (_H, _W, _C))
    row_b = jnp.broadcast_to(row_ref[...][:, None, :], (_H, _W, _C))
    for k in range(_KREP):
        scratch[k, :, :, :_C] = col_b
        scratch[k, :, :, _C:] = row_b
    n = _B // _KREP
    for i in range(n):
        pltpu.make_async_copy(
            scratch, out_ref.at[pl.ds(i * _KREP, _KREP)], sems.at[i]).start()
    for i in range(n):
        pltpu.make_async_copy(
            scratch, out_ref.at[pl.ds(i * _KREP, _KREP)], sems.at[i]).wait()


def kernel(mask, row_embed, col_embed):
    b = mask.shape[0]
    h, w = mask.shape[-2], mask.shape[-1]
    c = row_embed.shape[-1]
    out = pl.pallas_call(
        _body,
        grid=(1,),
        in_specs=[
            pl.BlockSpec((h, c), lambda i: (0, 0)),
            pl.BlockSpec((w, c), lambda i: (0, 0)),
        ],
        out_specs=pl.BlockSpec(memory_space=pl.ANY),
        out_shape=jax.ShapeDtypeStruct((b, h, w, 2 * c), jnp.float32),
        scratch_shapes=[
            pltpu.VMEM((_KREP, h, w, 2 * c), jnp.float32),
            pltpu.SemaphoreType.DMA((b // _KREP,)),
        ],
    )(row_embed, col_embed)
    return jnp.transpose(out, (0, 3, 1, 2))
